# Initial kernel scaffold; baseline (speedup 1.0000x reference)
import jax
import jax.numpy as jnp
from jax.experimental import pallas as pl


def kernel(x, edge_index, W1_l, b1, W1_r, W2_l, b2, W2_r):
    raise NotImplementedError("write your pallas kernel here")


# speedup vs baseline: 6.7508x; 6.7508x over previous
import functools

import jax
import jax.numpy as jnp
from jax import lax
from jax.experimental import pallas as pl
from jax.experimental.pallas import tpu as pltpu
from jax.experimental.pallas import tpu_sc as plsc

N = 10000
E = 320000
D = 128

NC = 2
NS = 16
NT = NC * NS
EPT = E // NT
BLK = 80
NJ = EPT // BLK
NPAD = 10240
ROWS_PER_TILE = NPAD // NS

_mesh = plsc.VectorSubcoreMesh(core_axis_name="c", subcore_axis_name="s")


@functools.partial(
    pl.kernel,
    out_type=jax.ShapeDtypeStruct((NC, NPAD, D), jnp.float32),
    mesh=_mesh,
    scratch_types=[
        pltpu.VMEM((NJ, BLK), jnp.int32),
        pltpu.VMEM((BLK, D), jnp.float32),
        pltpu.VMEM_SHARED((NPAD, D), jnp.float32),
    ],
)
def _deg_kernel(dst3, zb, ones_h, degp, dst_v, ones_v, deg_sh):
    c = lax.axis_index("c")
    s = lax.axis_index("s")
    t = c * NS + s
    base = s * ROWS_PER_TILE
    pltpu.sync_copy(zb, deg_sh.at[pl.ds(base, ROWS_PER_TILE)])
    pltpu.sync_copy(dst3.at[t], dst_v)
    pltpu.sync_copy(ones_h, ones_v)
    plsc.subcore_barrier()

    def body(j, carry):
        pltpu.sync_copy(ones_v, deg_sh.at[dst_v.at[j]], add=True)
        return carry

    lax.fori_loop(0, NJ, body, 0)
    plsc.subcore_barrier()
    pltpu.sync_copy(deg_sh.at[pl.ds(base, ROWS_PER_TILE)],
                    degp.at[c, pl.ds(base, ROWS_PER_TILE)])


@functools.partial(
    pl.kernel,
    out_type=jax.ShapeDtypeStruct((NC, NPAD, D), jnp.float32),
    mesh=_mesh,
    scratch_types=[
        pltpu.VMEM((NJ, BLK), jnp.int32),
        pltpu.VMEM((NJ, BLK), jnp.int32),
        pltpu.VMEM((BLK, D), jnp.float32),
        pltpu.VMEM_SHARED((NPAD, D), jnp.float32),
        pltpu.SemaphoreType.DMA,
    ],
)
def _agg_kernel(h, src3, dst3, zb, aggp, src_v, dst_v, rows_v, agg_sh, sem):
    c = lax.axis_index("c")
    s = lax.axis_index("s")
    t = c * NS + s
    base = s * ROWS_PER_TILE
    pltpu.sync_copy(zb, agg_sh.at[pl.ds(base, ROWS_PER_TILE)])
    pltpu.sync_copy(src3.at[t], src_v)
    pltpu.sync_copy(dst3.at[t], dst_v)
    plsc.subcore_barrier()

    def body(j, carry):
        pltpu.async_copy(h.at[src_v.at[j]], rows_v, sem).wait()
        pltpu.sync_copy(rows_v, agg_sh.at[dst_v.at[j]], add=True)
        return carry

    lax.fori_loop(0, NJ, body, 0)
    plsc.subcore_barrier()
    pltpu.sync_copy(agg_sh.at[pl.ds(base, ROWS_PER_TILE)],
                    aggp.at[c, pl.ds(base, ROWS_PER_TILE)])


R = 400


def _dense_body(a_ref, d_ref, h_ref, wl_ref, wr_ref, b_ref, o_ref, *, last):
    deg = jnp.maximum(d_ref[0, :, 0] + d_ref[1, :, 0], 1.0)
    agg = (a_ref[0] + a_ref[1]) / deg[:, None]
    z = (jnp.dot(agg, wl_ref[...], preferred_element_type=jnp.float32)
         + b_ref[...]
         + jnp.dot(h_ref[...], wr_ref[...], preferred_element_type=jnp.float32))
    if last:
        m = jnp.max(z, axis=-1, keepdims=True)
        lse = jnp.log(jnp.sum(jnp.exp(z - m), axis=-1, keepdims=True)) + m
        o_ref[...] = z - lse
    else:
        o_ref[...] = jnp.maximum(z, 0.0)


def _dense(aggp, degp, h, wl, wr, b, last):
    body = functools.partial(_dense_body, last=last)
    return pl.pallas_call(
        body,
        grid=(N // R,),
        in_specs=[
            pl.BlockSpec((NC, R, D), lambda i: (0, i, 0)),
            pl.BlockSpec((NC, R, D), lambda i: (0, i, 0)),
            pl.BlockSpec((R, D), lambda i: (i, 0)),
            pl.BlockSpec((D, D), lambda i: (0, 0)),
            pl.BlockSpec((D, D), lambda i: (0, 0)),
            pl.BlockSpec((1, D), lambda i: (0, 0)),
        ],
        out_specs=pl.BlockSpec((R, D), lambda i: (i, 0)),
        out_shape=jax.ShapeDtypeStruct((N, D), jnp.float32),
    )(aggp, degp, h, wl, wr, b)


@jax.jit
def kernel(x, edge_index, W1_l, b1, W1_r, W2_l, b2, W2_r):
    src3 = edge_index[0].reshape(NT, NJ, BLK)
    dst3 = edge_index[1].reshape(NT, NJ, BLK)
    zb = jnp.zeros((ROWS_PER_TILE, D), jnp.float32)
    ones_h = jnp.ones((BLK, D), jnp.float32)

    degp = _deg_kernel(dst3, zb, ones_h)
    aggp1 = _agg_kernel(x, src3, dst3, zb)
    h1 = _dense(aggp1, degp, x, W1_l, W1_r, b1.reshape(1, D), last=False)
    aggp2 = _agg_kernel(h1, src3, dst3, zb)
    out = _dense(aggp2, degp, h1, W2_l, W2_r, b2.reshape(1, D), last=True)
    return out


# speedup vs baseline: 7.6982x; 1.1403x over previous
import functools

import jax
import jax.numpy as jnp
from jax import lax
from jax.experimental import pallas as pl
from jax.experimental.pallas import tpu as pltpu
from jax.experimental.pallas import tpu_sc as plsc

N = 10000
E = 320000
D = 128

NC = 2
NS = 16
NT = NC * NS
EPT = E // NT
BLK = 80
NJ = EPT // BLK
NPAD = 10240
ROWS_PER_TILE = NPAD // NS

_mesh = plsc.VectorSubcoreMesh(core_axis_name="c", subcore_axis_name="s")


@functools.partial(
    pl.kernel,
    out_type=jax.ShapeDtypeStruct((NT, NPAD), jnp.float32),
    mesh=_mesh,
    scratch_types=[
        pltpu.VMEM((EPT,), jnp.int32),
        pltpu.VMEM((NPAD,), jnp.float32),
    ],
    compiler_params=pltpu.CompilerParams(needs_layout_passes=False),
)
def _deg_kernel(dst2, degp, dst_v, deg_v):
    c = lax.axis_index("c")
    s = lax.axis_index("s")
    t = c * NS + s
    pltpu.sync_copy(dst2.at[t], dst_v)

    def zbody(i, carry):
        deg_v[pl.ds(i * 16, 16)] = jnp.zeros((16,), jnp.float32)
        return carry

    lax.fori_loop(0, NPAD // 16, zbody, 0)
    ones = jnp.ones((16,), jnp.float32)

    def body(j, carry):
        idx = dst_v[pl.ds(j * 16, 16)]
        plsc.addupdate_scatter(deg_v, [idx], ones)
        return carry

    lax.fori_loop(0, EPT // 16, body, 0)
    pltpu.sync_copy(deg_v, degp.at[t])


@functools.partial(
    pl.kernel,
    out_type=jax.ShapeDtypeStruct((NC, NPAD, D), jnp.float32),
    mesh=_mesh,
    scratch_types=[
        pltpu.VMEM((NJ, BLK), jnp.int32),
        pltpu.VMEM((NJ, BLK), jnp.int32),
        pltpu.VMEM((BLK, D), jnp.float32),
        pltpu.VMEM_SHARED((NPAD, D), jnp.float32),
        pltpu.SemaphoreType.DMA,
    ],
)
def _agg_kernel(h, src3, dst3, zb, aggp, src_v, dst_v, rows_v, agg_sh, sem):
    c = lax.axis_index("c")
    s = lax.axis_index("s")
    t = c * NS + s
    base = s * ROWS_PER_TILE
    pltpu.sync_copy(zb, agg_sh.at[pl.ds(base, ROWS_PER_TILE)])
    pltpu.sync_copy(src3.at[t], src_v)
    pltpu.sync_copy(dst3.at[t], dst_v)
    plsc.subcore_barrier()

    def body(j, carry):
        pltpu.async_copy(h.at[src_v.at[j]], rows_v, sem).wait()
        pltpu.sync_copy(rows_v, agg_sh.at[dst_v.at[j]], add=True)
        return carry

    lax.fori_loop(0, NJ, body, 0)
    plsc.subcore_barrier()
    pltpu.sync_copy(agg_sh.at[pl.ds(base, ROWS_PER_TILE)],
                    aggp.at[c, pl.ds(base, ROWS_PER_TILE)])


R = 400


def _dense_body(a_ref, d_ref, h_ref, wl_ref, wr_ref, b_ref, o_ref, *, last):
    deg = jnp.maximum(jnp.sum(d_ref[0], axis=0), 1.0)
    agg = (a_ref[0] + a_ref[1]) / deg[:, None]
    z = (jnp.dot(agg, wl_ref[...], preferred_element_type=jnp.float32)
         + b_ref[...]
         + jnp.dot(h_ref[...], wr_ref[...], preferred_element_type=jnp.float32))
    if last:
        m = jnp.max(z, axis=-1, keepdims=True)
        lse = jnp.log(jnp.sum(jnp.exp(z - m), axis=-1, keepdims=True)) + m
        o_ref[...] = z - lse
    else:
        o_ref[...] = jnp.maximum(z, 0.0)


def _dense(aggp, degp, h, wl, wr, b, last):
    body = functools.partial(_dense_body, last=last)
    return pl.pallas_call(
        body,
        grid=(N // R,),
        in_specs=[
            pl.BlockSpec((NC, R, D), lambda i: (0, i, 0)),
            pl.BlockSpec((1, NT, R), lambda i: (i, 0, 0)),
            pl.BlockSpec((R, D), lambda i: (i, 0)),
            pl.BlockSpec((D, D), lambda i: (0, 0)),
            pl.BlockSpec((D, D), lambda i: (0, 0)),
            pl.BlockSpec((1, D), lambda i: (0, 0)),
        ],
        out_specs=pl.BlockSpec((R, D), lambda i: (i, 0)),
        out_shape=jax.ShapeDtypeStruct((N, D), jnp.float32),
    )(aggp, degp, h, wl, wr, b)


@jax.jit
def kernel(x, edge_index, W1_l, b1, W1_r, W2_l, b2, W2_r):
    src3 = edge_index[0].reshape(NT, NJ, BLK)
    dst3 = edge_index[1].reshape(NT, NJ, BLK)
    dst2 = edge_index[1].reshape(NT, EPT)
    zb = jnp.zeros((ROWS_PER_TILE, D), jnp.float32)

    degp = _deg_kernel(dst2)
    degt = degp[:, :N].reshape(NT, N // R, R).transpose(1, 0, 2)
    aggp1 = _agg_kernel(x, src3, dst3, zb)
    h1 = _dense(aggp1, degt, x, W1_l, W1_r, b1.reshape(1, D), last=False)
    aggp2 = _agg_kernel(h1, src3, dst3, zb)
    out = _dense(aggp2, degt, h1, W2_l, W2_r, b2.reshape(1, D), last=True)
    return out
